# baseline (speedup 1.0000x reference)
import functools

import jax
import jax.numpy as jnp
from jax.experimental import pallas as pl
from jax.experimental.pallas import tpu as pltpu


def _ln(x, w, b):
    m = jnp.mean(x, axis=-1, keepdims=True)
    v = jnp.mean((x - m) ** 2, axis=-1, keepdims=True)
    return (x - m) / jnp.sqrt(v + 1e-5) * w + b


def _angle(v1, v2):
    c = jnp.cross(v1, v2)
    return jnp.arctan2(
        jnp.sqrt(jnp.sum(c * c, axis=-1) + 1e-12), jnp.sum(v1 * v2, axis=-1)
    )


def _atom_mlp_kernel(x_ref, wa_ref, ba_ref, lw_ref, lb_ref, o_ref):
    x = x_ref[...]
    h = jax.nn.relu(
        jax.lax.dot_general(
            x, wa_ref[...], (((1,), (0,)), ((), ())),
            preferred_element_type=jnp.float32,
        )
        + ba_ref[...]
    )
    m = jnp.mean(h, axis=-1, keepdims=True)
    v = jnp.mean((h - m) ** 2, axis=-1, keepdims=True)
    o_ref[...] = (h - m) * jax.lax.rsqrt(v + 1e-5) * lw_ref[...] + lb_ref[...]


def _atom_mlp(x, Wa, ba, lw, lb, blk):
    n, din = x.shape
    dout = Wa.shape[1]
    grid = (n // blk,)
    return pl.pallas_call(
        _atom_mlp_kernel,
        grid=grid,
        in_specs=[
            pl.BlockSpec((blk, din), lambda i: (i, 0)),
            pl.BlockSpec((din, dout), lambda i: (0, 0)),
            pl.BlockSpec((dout,), lambda i: (0,)),
            pl.BlockSpec((dout,), lambda i: (0,)),
            pl.BlockSpec((dout,), lambda i: (0,)),
        ],
        out_specs=pl.BlockSpec((blk, dout), lambda i: (i, 0)),
        out_shape=jax.ShapeDtypeStruct((n, dout), jnp.float32),
    )(x, Wa, ba, lw, lb)


def kernel(pos_A, normal_A, pos_B, normal_B, conv_W1, conv_b1, conv_ln1_w,
           conv_ln1_b, conv_W2, conv_b2, conv_ln2_w, conv_ln2_b, Wa, ba,
           lna_w, lna_b, Wr, br, lnr_w, lnr_b, Wl, bl, edge_index_A,
           edge_index_B, residue_ids_A, residue_ids_B, src_res_idx,
           tgt_res_idx):
    n_nodes = pos_A.shape[0]
    n_res = 1000

    def side(pos, normal, edge_index, res_ids):
        src = edge_index[0]
        dst = edge_index[1]
        d = pos[src] - pos[dst]
        dn = jnp.sqrt(jnp.sum(d * d, axis=-1) + 1e-12)
        ni = normal[dst]
        nj = normal[src]
        ppf = jnp.stack([dn, _angle(ni, d), _angle(nj, d), _angle(ni, nj)], axis=1)
        feats = []
        for i in range(3):
            h = _ln(jax.nn.relu(ppf @ conv_W1[i] + conv_b1[i]), conv_ln1_w[i],
                    conv_ln1_b[i])
            h = _ln(jax.nn.relu(h @ conv_W2[i] + conv_b2[i]), conv_ln2_w[i],
                    conv_ln2_b[i])
            agg = jax.ops.segment_max(h, dst, num_segments=n_nodes)
            feats.append(jnp.where(jnp.isneginf(agg), 0.0, agg))
        atom = _atom_mlp(jnp.concatenate(feats, axis=1), Wa, ba, lna_w, lna_b,
                         blk=1000)
        res = jax.ops.segment_max(atom, res_ids, num_segments=n_res)
        res = jnp.where(jnp.isneginf(res), 0.0, res)
        return _atom_mlp(res, Wr, br, lnr_w, lnr_b, blk=1000)

    res_A = side(pos_A, normal_A, edge_index_A, residue_ids_A)
    res_B = side(pos_B, normal_B, edge_index_B, residue_ids_B)
    x_s = res_A[src_res_idx]
    x_t = res_B[tgt_res_idx]
    out = jax.nn.sigmoid((x_s * x_t) @ Wl + bl)[:, 0]
    return out


# speedup vs baseline: 2.1624x; 2.1624x over previous
import functools

import jax
import jax.numpy as jnp
from jax import lax
from jax.experimental import pallas as pl
from jax.experimental.pallas import tpu as pltpu
from jax.experimental.pallas import tpu_sc as plsc

N_NODES = 10000
N_EDGES = 320000
N_RES = 1000
NC, NS, LANES = 2, 16, 16
NW = NC * NS

_GCH = 2560
_NCHUNKS = N_EDGES // _GCH


def _sc_edge_gather(table_flat, src, dst):
    mesh = plsc.VectorSubcoreMesh(core_axis_name="c", subcore_axis_name="s")

    @functools.partial(
        pl.kernel,
        out_type=jax.ShapeDtypeStruct((16, N_EDGES), jnp.float32),
        mesh=mesh,
        scratch_types=[
            pltpu.VMEM((N_NODES * 8,), jnp.float32),
            pltpu.VMEM((_GCH,), jnp.int32),
            pltpu.VMEM((_GCH,), jnp.int32),
            pltpu.VMEM((16 * _GCH,), jnp.float32),
        ],
        compiler_params=pltpu.CompilerParams(needs_layout_passes=False),
    )
    def k(tab_hbm, src_hbm, dst_hbm, out_hbm, tab_v, si_v, di_v, gb_v):
        wid = lax.axis_index("s") * NC + lax.axis_index("c")
        pltpu.sync_copy(tab_hbm, tab_v)
        nch = jnp.where(wid < _NCHUNKS - NW * (_NCHUNKS // NW),
                        _NCHUNKS // NW + 1, _NCHUNKS // NW)

        def chunk(i, carry):
            base = (wid + i * NW) * _GCH
            pltpu.sync_copy(src_hbm.at[pl.ds(base, _GCH)], si_v)
            pltpu.sync_copy(dst_hbm.at[pl.ds(base, _GCH)], di_v)

            def grp(g, c2):
                s16 = si_v[pl.ds(g * 16, 16)] * 8
                d16 = di_v[pl.ds(g * 16, 16)] * 8
                for c in range(3):
                    gb_v[pl.ds(c * _GCH + g * 16, 16)] = plsc.load_gather(
                        tab_v, [s16 + c])
                    gb_v[pl.ds((3 + c) * _GCH + g * 16, 16)] = \
                        plsc.load_gather(tab_v, [d16 + c])
                    gb_v[pl.ds((6 + c) * _GCH + g * 16, 16)] = \
                        plsc.load_gather(tab_v, [s16 + 3 + c])
                    gb_v[pl.ds((9 + c) * _GCH + g * 16, 16)] = \
                        plsc.load_gather(tab_v, [d16 + 3 + c])
                return c2

            lax.fori_loop(0, _GCH // 16, grp, 0)
            for c in range(12):
                pltpu.sync_copy(gb_v.at[pl.ds(c * _GCH, _GCH)],
                                out_hbm.at[c, pl.ds(base, _GCH)])
            return carry

        lax.fori_loop(0, nch, chunk, 0)

    return k(table_flat, src, dst)


_EBLK = 2560


def _edge_mlp_kernel(g_ref, w1s_ref, b1_ref, lw1_ref, lb1_ref, w2s_ref,
                     b2_ref, lw2_ref, lb2_ref, o_ref):
    g = g_ref[...]
    d = g[0:3, :] - g[3:6, :]
    nj = g[6:9, :]
    ni = g[9:12, :]
    dn = jnp.sqrt(jnp.sum(d * d, axis=0, keepdims=True) + 1e-12)

    def ang(v1, v2):
        cx = v1[1:2, :] * v2[2:3, :] - v1[2:3, :] * v2[1:2, :]
        cy = v1[2:3, :] * v2[0:1, :] - v1[0:1, :] * v2[2:3, :]
        cz = v1[0:1, :] * v2[1:2, :] - v1[1:2, :] * v2[0:1, :]
        cn = jnp.sqrt(cx * cx + cy * cy + cz * cz + 1e-12)
        dt = jnp.sum(v1 * v2, axis=0, keepdims=True)
        return jnp.arctan2(cn, dt)

    ppf = jnp.concatenate([dn, ang(ni, d), ang(nj, d), ang(ni, nj)], axis=0)
    ones4 = jnp.ones((1, 4), jnp.float32)
    ones128 = jnp.ones((1, 128), jnp.float32)
    for i in range(3):
        w1t = w1s_ref[4 * i:4 * i + 4, :]
        p = jax.nn.relu(
            lax.dot_general(w1t, ppf, (((1,), (0,)), ((), ())),
                            preferred_element_type=jnp.float32)
            + b1_ref[4 * i:4 * i + 4, :])
        m = lax.dot_general(ones4, p, (((1,), (0,)), ((), ())),
                            preferred_element_type=jnp.float32) * 0.25
        pc = p - m
        v = lax.dot_general(ones4, pc * pc, (((1,), (0,)), ((), ())),
                            preferred_element_type=jnp.float32) * 0.25
        h1 = pc / jnp.sqrt(v + 1e-5) * lw1_ref[4 * i:4 * i + 4, :] \
            + lb1_ref[4 * i:4 * i + 4, :]
        w2t = w2s_ref[128 * i:128 * i + 128, :]
        q = jax.nn.relu(
            lax.dot_general(w2t, h1, (((1,), (0,)), ((), ())),
                            preferred_element_type=jnp.float32)
            + b2_ref[128 * i:128 * i + 128, :])
        m2 = lax.dot_general(ones128, q, (((1,), (0,)), ((), ())),
                             preferred_element_type=jnp.float32) * (1.0 / 128.0)
        qc = q - m2
        v2 = lax.dot_general(ones128, qc * qc, (((1,), (0,)), ((), ())),
                             preferred_element_type=jnp.float32) * (1.0 / 128.0)
        o_ref[128 * i:128 * i + 128, :] = (
            qc / jnp.sqrt(v2 + 1e-5) * lw2_ref[128 * i:128 * i + 128, :]
            + lb2_ref[128 * i:128 * i + 128, :])


def _edge_mlp(g, w1s, b1c, lw1c, lb1c, w2s, b2c, lw2c, lb2c):
    grid = (N_EDGES // _EBLK,)
    wspec = lambda r: pl.BlockSpec((r, 1), lambda i: (0, 0))
    return pl.pallas_call(
        _edge_mlp_kernel,
        grid=grid,
        in_specs=[
            pl.BlockSpec((16, _EBLK), lambda i: (0, i)),
            pl.BlockSpec((12, 4), lambda i: (0, 0)),
            wspec(12), wspec(12), wspec(12),
            pl.BlockSpec((384, 4), lambda i: (0, 0)),
            wspec(384), wspec(384), wspec(384),
        ],
        out_specs=pl.BlockSpec((384, _EBLK), lambda i: (0, i)),
        out_shape=jax.ShapeDtypeStruct((384, N_EDGES), jnp.float32),
    )(g, w1s, b1c, lw1c, lb1c, w2s, b2c, lw2c, lb2c)


def _mlp_ln_kernel(x_ref, w_ref, b_ref, lw_ref, lb_ref, o_ref):
    h = jax.nn.relu(
        lax.dot_general(x_ref[...], w_ref[...], (((1,), (0,)), ((), ())),
                        preferred_element_type=jnp.float32) + b_ref[...])
    m = jnp.mean(h, axis=-1, keepdims=True)
    v = jnp.mean((h - m) ** 2, axis=-1, keepdims=True)
    o_ref[...] = (h - m) / jnp.sqrt(v + 1e-5) * lw_ref[...] + lb_ref[...]


def _mlp_ln(x, W, b, lw, lb, blk):
    n, din = x.shape
    dout = W.shape[1]
    return pl.pallas_call(
        _mlp_ln_kernel,
        grid=(n // blk,),
        in_specs=[
            pl.BlockSpec((blk, din), lambda i: (i, 0)),
            pl.BlockSpec((din, dout), lambda i: (0, 0)),
            pl.BlockSpec((dout,), lambda i: (0,)),
            pl.BlockSpec((dout,), lambda i: (0,)),
            pl.BlockSpec((dout,), lambda i: (0,)),
        ],
        out_specs=pl.BlockSpec((blk, dout), lambda i: (i, 0)),
        out_shape=jax.ShapeDtypeStruct((n, dout), jnp.float32),
    )(x, W, b, lw, lb)


def kernel(pos_A, normal_A, pos_B, normal_B, conv_W1, conv_b1, conv_ln1_w,
           conv_ln1_b, conv_W2, conv_b2, conv_ln2_w, conv_ln2_b, Wa, ba,
           lna_w, lna_b, Wr, br, lnr_w, lnr_b, Wl, bl, edge_index_A,
           edge_index_B, residue_ids_A, residue_ids_B, src_res_idx,
           tgt_res_idx):
    f32 = jnp.float32
    w1s = jnp.transpose(conv_W1, (0, 2, 1)).reshape(12, 4)
    b1c = conv_b1.reshape(12, 1)
    lw1c = conv_ln1_w.reshape(12, 1)
    lb1c = conv_ln1_b.reshape(12, 1)
    w2s = jnp.transpose(conv_W2, (0, 2, 1)).reshape(384, 4)
    b2c = conv_b2.reshape(384, 1)
    lw2c = conv_ln2_w.reshape(384, 1)
    lb2c = conv_ln2_b.reshape(384, 1)

    def side(pos, normal, edge_index, res_ids):
        table = jnp.concatenate(
            [pos, normal, jnp.zeros((N_NODES, 2), f32)], axis=1).reshape(-1)
        src = edge_index[0]
        dst = edge_index[1]
        g = _sc_edge_gather(table, src, dst)
        h2t = _edge_mlp(g, w1s, b1c, lw1c, lb1c, w2s, b2c, lw2c, lb2c)
        feats = []
        for i in range(3):
            agg = jax.ops.segment_max(h2t[128 * i:128 * i + 128, :].T, dst,
                                      num_segments=N_NODES)
            feats.append(jnp.where(jnp.isneginf(agg), 0.0, agg))
        atom = _mlp_ln(jnp.concatenate(feats, axis=1), Wa, ba, lna_w, lna_b,
                       blk=1000)
        res = jax.ops.segment_max(atom, res_ids, num_segments=N_RES)
        res = jnp.where(jnp.isneginf(res), 0.0, res)
        return _mlp_ln(res, Wr, br, lnr_w, lnr_b, blk=1000)

    res_A = side(pos_A, normal_A, edge_index_A, residue_ids_A)
    res_B = side(pos_B, normal_B, edge_index_B, residue_ids_B)
    x_s = res_A[src_res_idx]
    x_t = res_B[tgt_res_idx]
    out = jax.nn.sigmoid((x_s * x_t) @ Wl + bl)[:, 0]
    return out
